# Initial kernel scaffold; baseline (speedup 1.0000x reference)
import functools

import jax
import jax.numpy as jnp
from jax import lax
from jax.experimental import pallas as pl
from jax.experimental.pallas import tpu as pltpu

D_MODEL = 1024
POOL = 64
TOPK = 8
D_FF = 2048
MOD = 64
HEADS = 16
DH = D_MODEL // HEADS
S = 2048

BS = 256
QB = 256
NEG = -1e30


def _dot(a, b):
    return jax.lax.dot_general(a, b, (((1,), (0,)), ((), ())),
                               preferred_element_type=jnp.float32)


def _dot_t(a, b):
    return jax.lax.dot_general(a, b, (((1,), (1,)), ((), ())),
                               preferred_element_type=jnp.float32)


def _routing_weights(scores):
    bs = scores.shape[0]
    iota = lax.broadcasted_iota(jnp.int32, (bs, POOL), 1)
    work = scores
    sel = jnp.zeros(scores.shape, dtype=jnp.bool_)
    for _ in range(TOPK):
        m = jnp.max(work, axis=-1, keepdims=True)
        cand = work == m
        first = jnp.min(jnp.where(cand, iota, POOL), axis=-1, keepdims=True)
        pick = iota == first
        sel = jnp.logical_or(sel, pick)
        work = jnp.where(pick, NEG, work)
    m0 = jnp.max(scores, axis=-1, keepdims=True)
    e = jnp.where(sel, jnp.exp(scores - m0), 0.0)
    return e / jnp.sum(e, axis=-1, keepdims=True)


def _pool_body(x_ref, wr_ref, br_ref, wc_ref, bc_ref, me_ref, pat_ref,
               wi_ref, bi_ref, wo_ref, bo_ref, no_ref):
    x = x_ref[...]
    scores = _dot(x, wr_ref[...]) + br_ref[...]
    pw = _routing_weights(scores)
    ctx = _dot(x, wc_ref[...]) + bc_ref[...]
    mod_scale = jax.nn.sigmoid(_dot_t(ctx, me_ref[...]))
    w = pw * mod_scale
    h = jax.nn.gelu(_dot(x, wi_ref[...]) + bi_ref[...])
    wp = _dot(w, pat_ref[...])
    fired = h * wp
    no_ref[...] = _dot(fired, wo_ref[...]) + bo_ref[...]


def _qkv_body(no_ref, wq_ref, bq_ref, qkv_ref):
    qkv_ref[...] = _dot(no_ref[...], wq_ref[...]) + bq_ref[...]


def _attn_body(q_ref, k_ref, v_ref, wo_ref, bo_ref, x_ref, no_ref,
               ln1g_ref, ln1b_ref, ln2g_ref, ln2b_ref, out_ref, acc_ref):
    h = pl.program_id(1)
    s = _dot_t(q_ref[...], k_ref[...]) * (1.0 / (DH ** 0.5))
    m = jnp.max(s, axis=-1, keepdims=True)
    p = jnp.exp(s - m)
    l = jnp.sum(p, axis=-1, keepdims=True)
    ao = _dot(p, v_ref[...]) / l

    @pl.when(h == 0)
    def _():
        acc_ref[...] = jnp.zeros_like(acc_ref)

    acc_ref[...] += _dot(ao, wo_ref[...])

    @pl.when(h == HEADS - 1)
    def _():
        attn_out = acc_ref[...] + bo_ref[...]
        t = x_ref[...] + attn_out
        mu = jnp.mean(t, axis=-1, keepdims=True)
        var = jnp.mean((t - mu) ** 2, axis=-1, keepdims=True)
        x1 = (t - mu) * lax.rsqrt(var + 1e-5) * ln1g_ref[...] + ln1b_ref[...]
        t2 = x1 + no_ref[...]
        mu2 = jnp.mean(t2, axis=-1, keepdims=True)
        var2 = jnp.mean((t2 - mu2) ** 2, axis=-1, keepdims=True)
        out_ref[...] = ((t2 - mu2) * lax.rsqrt(var2 + 1e-5) * ln2g_ref[...]
                        + ln2b_ref[...])


@jax.jit
def _run(x, W_router, b_router, patterns, mod_emb, W_ctx, b_ctx,
         W_in, b_in, W_out, b_out, W_qkv, b_qkv, W_o, b_o,
         ln1_g, ln1_b, ln2_g, ln2_b):
    x2 = x.reshape(S, D_MODEL)
    row = lambda v: v.reshape(1, -1)

    full = lambda shape: pl.BlockSpec(shape, lambda i: (0, 0))
    seq = lambda w: pl.BlockSpec((BS, w), lambda i: (i, 0))

    neuron_output = pl.pallas_call(
        _pool_body,
        grid=(S // BS,),
        in_specs=[
            seq(D_MODEL),
            full((D_MODEL, POOL)),
            full((1, POOL)),
            full((D_MODEL, MOD)),
            full((1, MOD)),
            full((POOL, MOD)),
            full((POOL, D_FF)),
            full((D_MODEL, D_FF)),
            full((1, D_FF)),
            full((D_FF, D_MODEL)),
            full((1, D_MODEL)),
        ],
        out_specs=seq(D_MODEL),
        out_shape=jax.ShapeDtypeStruct((S, D_MODEL), jnp.float32),
        compiler_params=pltpu.CompilerParams(
            dimension_semantics=("arbitrary",)),
    )(x2, W_router, row(b_router), W_ctx, row(b_ctx), mod_emb, patterns,
      W_in, row(b_in), W_out, row(b_out))

    qkv = pl.pallas_call(
        _qkv_body,
        grid=(S // BS,),
        in_specs=[seq(D_MODEL), full((D_MODEL, 3 * D_MODEL)),
                  full((1, 3 * D_MODEL))],
        out_specs=seq(3 * D_MODEL),
        out_shape=jax.ShapeDtypeStruct((S, 3 * D_MODEL), jnp.float32),
        compiler_params=pltpu.CompilerParams(
            dimension_semantics=("arbitrary",)),
    )(neuron_output, W_qkv, row(b_qkv))

    out = pl.pallas_call(
        _attn_body,
        grid=(S // QB, HEADS),
        in_specs=[
            pl.BlockSpec((QB, DH), lambda i, h: (i, h)),
            pl.BlockSpec((S, DH), lambda i, h: (0, HEADS + h)),
            pl.BlockSpec((S, DH), lambda i, h: (0, 2 * HEADS + h)),
            pl.BlockSpec((DH, D_MODEL), lambda i, h: (h, 0)),
            pl.BlockSpec((1, D_MODEL), lambda i, h: (0, 0)),
            pl.BlockSpec((QB, D_MODEL), lambda i, h: (i, 0)),
            pl.BlockSpec((QB, D_MODEL), lambda i, h: (i, 0)),
            pl.BlockSpec((1, D_MODEL), lambda i, h: (0, 0)),
            pl.BlockSpec((1, D_MODEL), lambda i, h: (0, 0)),
            pl.BlockSpec((1, D_MODEL), lambda i, h: (0, 0)),
            pl.BlockSpec((1, D_MODEL), lambda i, h: (0, 0)),
        ],
        out_specs=pl.BlockSpec((QB, D_MODEL), lambda i, h: (i, 0)),
        out_shape=jax.ShapeDtypeStruct((S, D_MODEL), jnp.float32),
        scratch_shapes=[pltpu.VMEM((QB, D_MODEL), jnp.float32)],
        compiler_params=pltpu.CompilerParams(
            dimension_semantics=("parallel", "arbitrary")),
    )(qkv, qkv, qkv, W_o, row(b_o), x2, neuron_output,
      row(ln1_g), row(ln1_b), row(ln2_g), row(ln2_b))

    return out.reshape(1, S, D_MODEL)


def kernel(x, W_router, b_router, patterns, mod_emb, W_ctx, b_ctx, W_in,
           b_in, W_out, b_out, W_qkv, b_qkv, W_o, b_o, ln1_g, ln1_b,
           ln2_g, ln2_b):
    return _run(x, W_router, b_router, patterns, mod_emb, W_ctx, b_ctx,
                W_in, b_in, W_out, b_out, W_qkv, b_qkv, W_o, b_o,
                ln1_g, ln1_b, ln2_g, ln2_b)


# speedup vs baseline: 2.0288x; 2.0288x over previous
import functools

import jax
import jax.numpy as jnp
from jax import lax
from jax.experimental import pallas as pl
from jax.experimental.pallas import tpu as pltpu

D_MODEL = 1024
POOL = 64
TOPK = 8
D_FF = 2048
MOD = 64
HEADS = 16
DH = D_MODEL // HEADS
S = 2048

BS = 256
QB = 256
NEG = -1e30


def _dot(a, b):
    return jax.lax.dot_general(a, b, (((1,), (0,)), ((), ())),
                               preferred_element_type=jnp.float32)


def _dot_t(a, b):
    return jax.lax.dot_general(a, b, (((1,), (1,)), ((), ())),
                               preferred_element_type=jnp.float32)


def _routing_weights(scores):
    bs = scores.shape[0]
    iota = lax.broadcasted_iota(jnp.int32, (bs, POOL), 1)
    work = scores
    sel = jnp.zeros(scores.shape, dtype=jnp.bool_)
    for _ in range(TOPK):
        m = jnp.max(work, axis=-1, keepdims=True)
        cand = work == m
        first = jnp.min(jnp.where(cand, iota, POOL), axis=-1, keepdims=True)
        pick = iota == first
        sel = jnp.logical_or(sel, pick)
        work = jnp.where(pick, NEG, work)
    m0 = jnp.max(scores, axis=-1, keepdims=True)
    e = jnp.where(sel, jnp.exp(scores - m0), 0.0)
    return e / jnp.sum(e, axis=-1, keepdims=True)


def _pool_body(x_ref, wr_ref, br_ref, wc_ref, bc_ref, me_ref, pat_ref,
               wi_ref, bi_ref, wo_ref, bo_ref, no_ref):
    x = x_ref[...]
    scores = _dot(x, wr_ref[...]) + br_ref[...]
    pw = _routing_weights(scores)
    ctx = _dot(x, wc_ref[...]) + bc_ref[...]
    mod_scale = jax.nn.sigmoid(_dot_t(ctx, me_ref[...]))
    w = pw * mod_scale
    h = jax.nn.gelu(_dot(x, wi_ref[...]) + bi_ref[...])
    wp = _dot(w, pat_ref[...])
    fired = h * wp
    no_ref[...] = _dot(fired, wo_ref[...]) + bo_ref[...]


def _qkv_body(no_ref, wq_ref, bq_ref, qkv_ref):
    qkv_ref[...] = _dot(no_ref[...], wq_ref[...]) + bq_ref[...]


def _attn_body(q_ref, k_ref, v_ref, wo_ref, bo_ref, x_ref, no_ref,
               ln1g_ref, ln1b_ref, ln2g_ref, ln2b_ref, out_ref):
    q_all = q_ref[...]
    k_all = k_ref[...]
    v_all = v_ref[...]
    wo = wo_ref[...]
    acc = jnp.zeros((q_all.shape[0], D_MODEL), jnp.float32)
    for h in range(HEADS):
        sl = slice(h * DH, (h + 1) * DH)
        s = _dot_t(q_all[:, sl], k_all[:, sl]) * (1.0 / (DH ** 0.5))
        m = jnp.max(s, axis=-1, keepdims=True)
        p = jnp.exp(s - m)
        l = jnp.sum(p, axis=-1, keepdims=True)
        ao = _dot(p, v_all[:, sl]) / l
        acc = acc + _dot(ao, wo[sl, :])

    attn_out = acc + bo_ref[...]
    t = x_ref[...] + attn_out
    mu = jnp.mean(t, axis=-1, keepdims=True)
    var = jnp.mean((t - mu) ** 2, axis=-1, keepdims=True)
    x1 = (t - mu) * lax.rsqrt(var + 1e-5) * ln1g_ref[...] + ln1b_ref[...]
    t2 = x1 + no_ref[...]
    mu2 = jnp.mean(t2, axis=-1, keepdims=True)
    var2 = jnp.mean((t2 - mu2) ** 2, axis=-1, keepdims=True)
    out_ref[...] = ((t2 - mu2) * lax.rsqrt(var2 + 1e-5) * ln2g_ref[...]
                    + ln2b_ref[...])


@jax.jit
def _run(x, W_router, b_router, patterns, mod_emb, W_ctx, b_ctx,
         W_in, b_in, W_out, b_out, W_qkv, b_qkv, W_o, b_o,
         ln1_g, ln1_b, ln2_g, ln2_b):
    x2 = x.reshape(S, D_MODEL)
    row = lambda v: v.reshape(1, -1)

    full = lambda shape: pl.BlockSpec(shape, lambda i: (0, 0))
    seq = lambda w: pl.BlockSpec((BS, w), lambda i: (i, 0))

    neuron_output = pl.pallas_call(
        _pool_body,
        grid=(S // BS,),
        in_specs=[
            seq(D_MODEL),
            full((D_MODEL, POOL)),
            full((1, POOL)),
            full((D_MODEL, MOD)),
            full((1, MOD)),
            full((POOL, MOD)),
            full((POOL, D_FF)),
            full((D_MODEL, D_FF)),
            full((1, D_FF)),
            full((D_FF, D_MODEL)),
            full((1, D_MODEL)),
        ],
        out_specs=seq(D_MODEL),
        out_shape=jax.ShapeDtypeStruct((S, D_MODEL), jnp.float32),
        compiler_params=pltpu.CompilerParams(
            dimension_semantics=("arbitrary",)),
    )(x2, W_router, row(b_router), W_ctx, row(b_ctx), mod_emb, patterns,
      W_in, row(b_in), W_out, row(b_out))

    qkv = pl.pallas_call(
        _qkv_body,
        grid=(S // BS,),
        in_specs=[seq(D_MODEL), full((D_MODEL, 3 * D_MODEL)),
                  full((1, 3 * D_MODEL))],
        out_specs=seq(3 * D_MODEL),
        out_shape=jax.ShapeDtypeStruct((S, 3 * D_MODEL), jnp.float32),
        compiler_params=pltpu.CompilerParams(
            dimension_semantics=("arbitrary",)),
    )(neuron_output, W_qkv, row(b_qkv))

    out = pl.pallas_call(
        _attn_body,
        grid=(S // QB,),
        in_specs=[
            pl.BlockSpec((QB, D_MODEL), lambda i: (i, 0)),
            pl.BlockSpec((S, D_MODEL), lambda i: (0, 1)),
            pl.BlockSpec((S, D_MODEL), lambda i: (0, 2)),
            full((D_MODEL, D_MODEL)),
            full((1, D_MODEL)),
            pl.BlockSpec((QB, D_MODEL), lambda i: (i, 0)),
            pl.BlockSpec((QB, D_MODEL), lambda i: (i, 0)),
            full((1, D_MODEL)),
            full((1, D_MODEL)),
            full((1, D_MODEL)),
            full((1, D_MODEL)),
        ],
        out_specs=pl.BlockSpec((QB, D_MODEL), lambda i: (i, 0)),
        out_shape=jax.ShapeDtypeStruct((S, D_MODEL), jnp.float32),
        compiler_params=pltpu.CompilerParams(
            dimension_semantics=("arbitrary",)),
    )(qkv, qkv, qkv, W_o, row(b_o), x2, neuron_output,
      row(ln1_g), row(ln1_b), row(ln2_g), row(ln2_b))

    return out.reshape(1, S, D_MODEL)


def kernel(x, W_router, b_router, patterns, mod_emb, W_ctx, b_ctx, W_in,
           b_in, W_out, b_out, W_qkv, b_qkv, W_o, b_o, ln1_g, ln1_b,
           ln2_g, ln2_b):
    return _run(x, W_router, b_router, patterns, mod_emb, W_ctx, b_ctx,
                W_in, b_in, W_out, b_out, W_qkv, b_qkv, W_o, b_o,
                ln1_g, ln1_b, ln2_g, ln2_b)


# speedup vs baseline: 2.2979x; 1.1326x over previous
import functools

import jax
import jax.numpy as jnp
from jax import lax
from jax.experimental import pallas as pl
from jax.experimental.pallas import tpu as pltpu

D_MODEL = 1024
POOL = 64
TOPK = 8
D_FF = 2048
MOD = 64
HEADS = 16
DH = D_MODEL // HEADS
S = 2048

BS = 256
QB = 256
NEG = -1e30


def _dot(a, b):
    return jax.lax.dot_general(a, b, (((1,), (0,)), ((), ())),
                               preferred_element_type=jnp.float32)


def _dot_t(a, b):
    return jax.lax.dot_general(a, b, (((1,), (1,)), ((), ())),
                               preferred_element_type=jnp.float32)


def _bf(t):
    return t.astype(jnp.bfloat16)


def _routing_weights(scores):
    bs = scores.shape[0]
    iota = lax.broadcasted_iota(jnp.int32, (bs, POOL), 1)
    work = scores
    sel = jnp.zeros(scores.shape, dtype=jnp.bool_)
    for _ in range(TOPK):
        m = jnp.max(work, axis=-1, keepdims=True)
        cand = work == m
        first = jnp.min(jnp.where(cand, iota, POOL), axis=-1, keepdims=True)
        pick = iota == first
        sel = jnp.logical_or(sel, pick)
        work = jnp.where(pick, NEG, work)
    m0 = jnp.max(scores, axis=-1, keepdims=True)
    e = jnp.where(sel, jnp.exp(scores - m0), 0.0)
    return e / jnp.sum(e, axis=-1, keepdims=True)


def _pool_body(x_ref, wr_ref, br_ref, wc_ref, bc_ref, me_ref, pat_ref,
               wi_ref, bi_ref, wo_ref, bo_ref, no_ref):
    x = x_ref[...]
    scores = _dot(x, wr_ref[...]) + br_ref[...]
    pw = _routing_weights(scores)
    ctx = _dot(x, wc_ref[...]) + bc_ref[...]
    mod_scale = jax.nn.sigmoid(_dot_t(ctx, me_ref[...]))
    w = pw * mod_scale
    h = jax.nn.gelu(_dot(_bf(x), wi_ref[...]) + bi_ref[...])
    wp = _dot(_bf(w), pat_ref[...])
    fired = h * wp
    no_ref[...] = _dot(_bf(fired), wo_ref[...]) + bo_ref[...]


def _qkv_body(no_ref, wq_ref, bq_ref, qkv_ref):
    qkv_ref[...] = _bf(_dot(_bf(no_ref[...]), wq_ref[...]) + bq_ref[...])


def _attn_body(q_ref, k_ref, v_ref, wo_ref, bo_ref, x_ref, no_ref,
               ln1g_ref, ln1b_ref, ln2g_ref, ln2b_ref, out_ref):
    q_all = q_ref[...]
    k_all = k_ref[...]
    v_all = v_ref[...]
    wo = wo_ref[...]
    acc = jnp.zeros((q_all.shape[0], D_MODEL), jnp.float32)
    for h in range(HEADS):
        sl = slice(h * DH, (h + 1) * DH)
        s = _dot_t(q_all[:, sl], k_all[:, sl]) * (1.0 / (DH ** 0.5))
        m = jnp.max(s, axis=-1, keepdims=True)
        p = jnp.exp(s - m)
        l = jnp.sum(p, axis=-1, keepdims=True)
        ao = _dot(_bf(p), v_all[:, sl]) / l
        acc = acc + _dot(_bf(ao), wo[sl, :])

    attn_out = acc + bo_ref[...]
    t = x_ref[...] + attn_out
    mu = jnp.mean(t, axis=-1, keepdims=True)
    var = jnp.mean((t - mu) ** 2, axis=-1, keepdims=True)
    x1 = (t - mu) * lax.rsqrt(var + 1e-5) * ln1g_ref[...] + ln1b_ref[...]
    t2 = x1 + no_ref[...]
    mu2 = jnp.mean(t2, axis=-1, keepdims=True)
    var2 = jnp.mean((t2 - mu2) ** 2, axis=-1, keepdims=True)
    out_ref[...] = ((t2 - mu2) * lax.rsqrt(var2 + 1e-5) * ln2g_ref[...]
                    + ln2b_ref[...])


@jax.jit
def _run(x, W_router, b_router, patterns, mod_emb, W_ctx, b_ctx,
         W_in, b_in, W_out, b_out, W_qkv, b_qkv, W_o, b_o,
         ln1_g, ln1_b, ln2_g, ln2_b):
    x2 = x.reshape(S, D_MODEL)
    row = lambda v: v.reshape(1, -1)
    W_in_b = W_in.astype(jnp.bfloat16)
    patterns_b = patterns.astype(jnp.bfloat16)
    W_out_b = W_out.astype(jnp.bfloat16)
    W_qkv_b = W_qkv.astype(jnp.bfloat16)
    W_o_b = W_o.astype(jnp.bfloat16)

    full = lambda shape: pl.BlockSpec(shape, lambda i: (0, 0))
    seq = lambda w: pl.BlockSpec((BS, w), lambda i: (i, 0))

    neuron_output = pl.pallas_call(
        _pool_body,
        grid=(S // BS,),
        in_specs=[
            seq(D_MODEL),
            full((D_MODEL, POOL)),
            full((1, POOL)),
            full((D_MODEL, MOD)),
            full((1, MOD)),
            full((POOL, MOD)),
            full((POOL, D_FF)),
            full((D_MODEL, D_FF)),
            full((1, D_FF)),
            full((D_FF, D_MODEL)),
            full((1, D_MODEL)),
        ],
        out_specs=seq(D_MODEL),
        out_shape=jax.ShapeDtypeStruct((S, D_MODEL), jnp.float32),
        compiler_params=pltpu.CompilerParams(
            dimension_semantics=("arbitrary",)),
    )(x2, W_router, row(b_router), W_ctx, row(b_ctx), mod_emb, patterns_b,
      W_in_b, row(b_in), W_out_b, row(b_out))

    qkv = pl.pallas_call(
        _qkv_body,
        grid=(S // BS,),
        in_specs=[seq(D_MODEL), full((D_MODEL, 3 * D_MODEL)),
                  full((1, 3 * D_MODEL))],
        out_specs=seq(3 * D_MODEL),
        out_shape=jax.ShapeDtypeStruct((S, 3 * D_MODEL), jnp.bfloat16),
        compiler_params=pltpu.CompilerParams(
            dimension_semantics=("arbitrary",)),
    )(neuron_output, W_qkv_b, row(b_qkv))

    out = pl.pallas_call(
        _attn_body,
        grid=(S // QB,),
        in_specs=[
            pl.BlockSpec((QB, D_MODEL), lambda i: (i, 0)),
            pl.BlockSpec((S, D_MODEL), lambda i: (0, 1)),
            pl.BlockSpec((S, D_MODEL), lambda i: (0, 2)),
            full((D_MODEL, D_MODEL)),
            full((1, D_MODEL)),
            pl.BlockSpec((QB, D_MODEL), lambda i: (i, 0)),
            pl.BlockSpec((QB, D_MODEL), lambda i: (i, 0)),
            full((1, D_MODEL)),
            full((1, D_MODEL)),
            full((1, D_MODEL)),
            full((1, D_MODEL)),
        ],
        out_specs=pl.BlockSpec((QB, D_MODEL), lambda i: (i, 0)),
        out_shape=jax.ShapeDtypeStruct((S, D_MODEL), jnp.float32),
        compiler_params=pltpu.CompilerParams(
            dimension_semantics=("arbitrary",)),
    )(qkv, qkv, qkv, W_o_b, row(b_o), x2, neuron_output,
      row(ln1_g), row(ln1_b), row(ln2_g), row(ln2_b))

    return out.reshape(1, S, D_MODEL)


def kernel(x, W_router, b_router, patterns, mod_emb, W_ctx, b_ctx, W_in,
           b_in, W_out, b_out, W_qkv, b_qkv, W_o, b_o, ln1_g, ln1_b,
           ln2_g, ln2_b):
    return _run(x, W_router, b_router, patterns, mod_emb, W_ctx, b_ctx,
                W_in, b_in, W_out, b_out, W_qkv, b_qkv, W_o, b_o,
                ln1_g, ln1_b, ln2_g, ln2_b)
